# baseline (speedup 1.0000x reference)
import functools

import jax
import jax.numpy as jnp
from jax.experimental import pallas as pl
from jax.experimental.pallas import tpu as pltpu

_INV_TEMP = 10.0
_EPS_SQ = 1e-24


def _matmul_kernel(x_ref, g_ref, out_ref):
    out_ref[...] = jax.lax.dot_general(
        x_ref[...], g_ref[...], (((1,), (1,)), ((), ())),
        preferred_element_type=jnp.float32)


@functools.partial(jax.jit, static_argnames=("bm",))
def _smog_logits(x, group_features, bm):
    b, d = x.shape
    k, _ = group_features.shape
    bm = min(bm, b)
    xs = x * (_INV_TEMP * jax.lax.rsqrt(
        jnp.maximum(jnp.sum(x * x, axis=1, keepdims=True), _EPS_SQ)))
    gs = group_features * jax.lax.rsqrt(
        jnp.maximum(jnp.sum(group_features * group_features, axis=1,
                            keepdims=True), _EPS_SQ))
    return pl.pallas_call(
        _matmul_kernel,
        grid=(b // bm,),
        in_specs=[
            pl.BlockSpec((bm, d), lambda i: (i, 0)),
            pl.BlockSpec((k, d), lambda i: (0, 0)),
        ],
        out_specs=pl.BlockSpec((bm, k), lambda i: (i, 0)),
        out_shape=jax.ShapeDtypeStruct((b, k), jnp.float32),
        compiler_params=pltpu.CompilerParams(
            dimension_semantics=("arbitrary",)),
    )(xs, gs)


def kernel(x, group_features):
    return _smog_logits(x, group_features, bm=256)
